# baseline (speedup 1.0000x reference)
import jax
import jax.numpy as jnp
from jax import lax
from jax.experimental import pallas as pl
from jax.experimental.pallas import tpu as pltpu
from jax.experimental.pallas import tpu_sc as plsc

N = 262144
M = 65536
C = 128
NW = 32
CHUNK = N // NW
SUB = 128
NSUB = CHUNK // SUB

VOXEL_SIZE = (1.0, 1.0, 0.08)
PC_MIN = (-50.0, -50.0, -5.0)

_LANES = 16


def _flags_body(pad_ref, feats_ref, flags_ref):
    pad = pad_ref[0, 0]
    flags_ref[...] = jnp.all(feats_ref[...] == pad, axis=1).astype(jnp.int32)


def _compute_flags(pad, voxel_feats):
    BM = 1024
    return pl.pallas_call(
        _flags_body,
        grid=(M // BM,),
        in_specs=[
            pl.BlockSpec(memory_space=pltpu.SMEM),
            pl.BlockSpec((BM, C), lambda i: (i, 0)),
        ],
        out_specs=pl.BlockSpec((BM,), lambda i: (i,)),
        out_shape=jax.ShapeDtypeStruct((M,), jnp.int32),
    )(pad, voxel_feats)


def _maskscan_body(flags_hbm, inds_hbm, mask_hbm, lcsum_hbm, totals_hbm,
                   flags_v, inds_v, mask_v, lcsum_v, tot16_v):
    cid = lax.axis_index("c")
    sid = lax.axis_index("s")
    wid = cid * 16 + sid
    base = wid * CHUNK
    lanes = jnp.arange(_LANES, dtype=jnp.int32)

    pltpu.sync_copy(flags_hbm, flags_v)
    pltpu.sync_copy(inds_hbm.at[pl.ds(base, CHUNK)], inds_v)

    def step(k, carry):
        idx = inds_v[pl.ds(k * _LANES, _LANES)]
        flg = plsc.load_gather(flags_v, [idx >> 7, idx & 127])
        m = 1 - flg
        mask_v[pl.ds(k * _LANES, _LANES)] = m
        lcsum_v[pl.ds(k * _LANES, _LANES)] = plsc.cumsum(m) + carry
        return carry + jnp.sum(m)

    total = lax.fori_loop(0, CHUNK // _LANES, step, jnp.int32(0))

    tot16_v[...] = jnp.where(lanes == 0, total, 0)
    pltpu.sync_copy(mask_v, mask_hbm.at[pl.ds(base, CHUNK)])
    pltpu.sync_copy(lcsum_v, lcsum_hbm.at[pl.ds(base, CHUNK)])
    pltpu.sync_copy(tot16_v, totals_hbm.at[wid])


def _mask_scan(flags, inds):
    mesh = plsc.VectorSubcoreMesh(core_axis_name="c", subcore_axis_name="s")
    return pl.kernel(
        _maskscan_body,
        compiler_params=pltpu.CompilerParams(needs_layout_passes=False, use_tc_tiling_on_sc=False),
        out_type=(
            jax.ShapeDtypeStruct((N,), jnp.int32),
            jax.ShapeDtypeStruct((N,), jnp.int32),
            jax.ShapeDtypeStruct((NW, _LANES), jnp.int32),
        ),
        mesh=mesh,
        scratch_types=[
            pltpu.VMEM((M // 128, 128), jnp.int32),
            pltpu.VMEM((CHUNK,), jnp.int32),
            pltpu.VMEM((CHUNK,), jnp.int32),
            pltpu.VMEM((CHUNK,), jnp.int32),
            pltpu.VMEM((_LANES,), jnp.int32),
        ],
    )(flags, inds)


def _main_body(vfeats_hbm, inds2_hbm, mask_hbm, lcsum_hbm, totals_hbm,
               px_hbm, py_hbm, pz_hbm, c1_hbm, c2_hbm, c3_hbm,
               feats_out, xs_out, ys_out, zs_out,
               mask_v, lcsum_v, inds2_v, dest2_v, tot_v,
               f0, f1, f2, f3, sl0, sl1, sl2, sl3,
               x0, x1, x2, x3, g0, g1, g2, g3, s0, s1, s2, s3):
    feats_bufs = (f0, f1, f2, f3)
    sl_bufs = (sl0, sl1, sl2, sl3)
    xst_bufs = (x0, x1, x2, x3)
    gsems = (g0, g1, g2, g3)
    ssems = (s0, s1, s2, s3)
    slice_srcs = (px_hbm, py_hbm, pz_hbm, c1_hbm, c2_hbm, c3_hbm)

    cid = lax.axis_index("c")
    sid = lax.axis_index("s")
    wid = cid * 16 + sid
    base = wid * CHUNK
    lanes = jnp.arange(_LANES, dtype=jnp.int32)
    zeros16 = jnp.zeros((_LANES,), jnp.int32)

    pltpu.sync_copy(mask_hbm.at[pl.ds(base, CHUNK)], mask_v)
    pltpu.sync_copy(lcsum_hbm.at[pl.ds(base, CHUNK)], lcsum_v)
    pltpu.sync_copy(inds2_hbm.at[pl.ds(wid * NSUB, NSUB)], inds2_v)
    pltpu.sync_copy(totals_hbm, tot_v)

    t_lo = plsc.load_gather(tot_v, [lanes, zeros16])
    t_hi = plsc.load_gather(tot_v, [lanes + 16, zeros16])
    zero_v = jnp.zeros((_LANES,), jnp.int32)
    my_pre = (jnp.sum(jnp.where(lanes < wid, t_lo, zero_v))
              + jnp.sum(jnp.where(lanes + 16 < wid, t_hi, zero_v)))
    tot_all = jnp.sum(t_lo) + jnp.sum(t_hi)

    @pl.loop(0, CHUNK // _LANES)
    def _dest(k):
        m = mask_v[pl.ds(k * _LANES, _LANES)]
        cs = lcsum_v[pl.ds(k * _LANES, _LANES)]
        gi = cs + my_pre
        ivec = base + k * _LANES + lanes
        d = jnp.where(m == 1, gi - 1, tot_all + ivec - gi)
        dest2_v[k >> 3, pl.ds((k & 7) * _LANES, _LANES)] = d

    def fire_gathers(j, b):
        pltpu.async_copy(vfeats_hbm.at[inds2_v.at[j]], feats_bufs[b], gsems[b])
        for r in range(6):
            pltpu.async_copy(slice_srcs[r].at[pl.ds(base + j * SUB, SUB)],
                             sl_bufs[b].at[r], gsems[b])

    def drain_gathers(b):
        pltpu.make_async_copy(vfeats_hbm.at[inds2_v.at[0]], feats_bufs[b],
                              gsems[b]).wait()
        for r in range(6):
            pltpu.make_async_copy(px_hbm.at[pl.ds(0, SUB)], sl_bufs[b].at[r],
                                  gsems[b]).wait()

    def fire_scatters(j, b):
        pltpu.async_copy(feats_bufs[b], feats_out.at[dest2_v.at[j]], ssems[b])
        pltpu.async_copy(xst_bufs[b].at[0], xs_out.at[dest2_v.at[j]], ssems[b])
        pltpu.async_copy(xst_bufs[b].at[1], ys_out.at[dest2_v.at[j]], ssems[b])
        pltpu.async_copy(xst_bufs[b].at[2], zs_out.at[dest2_v.at[j]], ssems[b])

    def drain_scatters(b):
        pltpu.make_async_copy(feats_bufs[b], feats_out.at[dest2_v.at[0]],
                              ssems[b]).wait()
        for r in range(3):
            pltpu.make_async_copy(xst_bufs[b].at[r],
                                  xs_out.at[dest2_v.at[0]], ssems[b]).wait()

    def compute(b):
        sl = sl_bufs[b]
        xst = xst_bufs[b]
        for grp in range(SUB // _LANES):
            s = grp * _LANES
            px = sl[0, pl.ds(s, _LANES)]
            py = sl[1, pl.ds(s, _LANES)]
            pz = sl[2, pl.ds(s, _LANES)]
            c1 = sl[3, pl.ds(s, _LANES)]
            c2 = sl[4, pl.ds(s, _LANES)]
            c3 = sl[5, pl.ds(s, _LANES)]
            cx = (c3 + 0.5) * VOXEL_SIZE[0] + PC_MIN[0]
            cy = (c2 + 0.5) * VOXEL_SIZE[1] + PC_MIN[1]
            cz = (c1 + 0.5) * VOXEL_SIZE[2] + PC_MIN[2]
            xst[0, pl.ds(s, _LANES)] = px - cx
            xst[1, pl.ds(s, _LANES)] = py - cy
            xst[2, pl.ds(s, _LANES)] = pz - cz

    fire_gathers(0, 0)
    fire_gathers(1, 1)

    @pl.loop(0, NSUB, step=4)
    def _ring(g):
        for b in range(4):
            j = g + b
            bp = (b + 2) % 4

            @pl.when(jnp.logical_and(j >= 2, j + 2 < NSUB))
            def _():
                drain_scatters(bp)

            @pl.when(j + 2 < NSUB)
            def _():
                fire_gathers(j + 2, bp)

            drain_gathers(b)
            compute(b)
            fire_scatters(j, b)

    drain_scatters(0)
    drain_scatters(1)
    drain_scatters(2)
    drain_scatters(3)


def _main_pass(voxel_feats, inds2, mask, lcsum, totals, cols):
    mesh = plsc.VectorSubcoreMesh(core_axis_name="c", subcore_axis_name="s")
    fbuf = pltpu.VMEM((SUB, C), jnp.float32)
    slbuf = pltpu.VMEM((6, SUB), jnp.float32)
    xbuf = pltpu.VMEM((3, SUB), jnp.float32)
    sem = pltpu.SemaphoreType.DMA
    return pl.kernel(
        _main_body,
        compiler_params=pltpu.CompilerParams(needs_layout_passes=False, use_tc_tiling_on_sc=False),
        out_type=(
            jax.ShapeDtypeStruct((N, C), jnp.float32),
            jax.ShapeDtypeStruct((N,), jnp.float32),
            jax.ShapeDtypeStruct((N,), jnp.float32),
            jax.ShapeDtypeStruct((N,), jnp.float32),
        ),
        mesh=mesh,
        scratch_types=(
            [pltpu.VMEM((CHUNK,), jnp.int32)] * 2
            + [pltpu.VMEM((NSUB, SUB), jnp.int32)] * 2
            + [pltpu.VMEM((NW, _LANES), jnp.int32)]
            + [fbuf] * 4 + [slbuf] * 4 + [xbuf] * 4 + [sem] * 8
        ),
    )(voxel_feats, inds2, mask, lcsum, totals, *cols)


def _assemble_body(f_ref, x_ref, y_ref, z_ref, o_ref):
    o_ref[...] = jnp.concatenate(
        [f_ref[...], x_ref[...][:, None], y_ref[...][:, None],
         z_ref[...][:, None]], axis=1)


def _assemble(feats_s, xs, ys, zs):
    BR = 1024
    return pl.pallas_call(
        _assemble_body,
        grid=(N // BR,),
        in_specs=[
            pl.BlockSpec((BR, C), lambda i: (i, 0)),
            pl.BlockSpec((BR,), lambda i: (i,)),
            pl.BlockSpec((BR,), lambda i: (i,)),
            pl.BlockSpec((BR,), lambda i: (i,)),
        ],
        out_specs=pl.BlockSpec((BR, C + 3), lambda i: (i, 0)),
        out_shape=jax.ShapeDtypeStruct((N, C + 3), jnp.float32),
    )(feats_s, xs, ys, zs)


def kernel(points, pts_coors, voxel_feats, voxel2point_inds, voxel_padding):
    pad = jnp.asarray(voxel_padding, jnp.float32).reshape(1, 1)
    flags = _compute_flags(pad, voxel_feats)
    mask_i32, lcsum, totals = _mask_scan(flags.reshape(M // 128, 128),
                                         voxel2point_inds)
    inds2 = voxel2point_inds.reshape(N // SUB, SUB)
    cols = (points[:, 0], points[:, 1], points[:, 2],
            pts_coors[:, 1].astype(jnp.float32),
            pts_coors[:, 2].astype(jnp.float32),
            pts_coors[:, 3].astype(jnp.float32))
    feats_s, xs, ys, zs = _main_pass(voxel_feats, inds2, mask_i32, lcsum,
                                     totals, cols)
    results = _assemble(feats_s, xs, ys, zs)
    return results, mask_i32.astype(bool)


# speedup vs baseline: 1.1283x; 1.1283x over previous
import jax
import jax.numpy as jnp
from jax import lax
from jax.experimental import pallas as pl
from jax.experimental.pallas import tpu as pltpu
from jax.experimental.pallas import tpu_sc as plsc

N = 262144
M = 65536
C = 128
NW = 32
CHUNK = N // NW
SUB = 128
NSUB = CHUNK // SUB

VOXEL_SIZE = (1.0, 1.0, 0.08)
PC_MIN = (-50.0, -50.0, -5.0)

_LANES = 16


def _flags_body(pad_ref, feats_ref, flags_ref):
    pad = pad_ref[0, 0]
    flags_ref[...] = jnp.all(feats_ref[...] == pad, axis=1).astype(jnp.int32)


def _compute_flags(pad, voxel_feats):
    BM = 1024
    return pl.pallas_call(
        _flags_body,
        grid=(M // BM,),
        in_specs=[
            pl.BlockSpec(memory_space=pltpu.SMEM),
            pl.BlockSpec((BM, C), lambda i: (i, 0)),
        ],
        out_specs=pl.BlockSpec((BM,), lambda i: (i,)),
        out_shape=jax.ShapeDtypeStruct((M,), jnp.int32),
    )(pad, voxel_feats)


def _maskscan_body(flags_hbm, inds_hbm, mask_hbm, lcsum_hbm, totals_hbm,
                   flags_v, inds_v, mask_v, lcsum_v, tot128_v):
    cid = lax.axis_index("c")
    sid = lax.axis_index("s")
    wid = cid * 16 + sid
    base = wid * CHUNK
    lanes = jnp.arange(_LANES, dtype=jnp.int32)

    pltpu.sync_copy(flags_hbm, flags_v)
    pltpu.sync_copy(inds_hbm.at[pl.ds(base, CHUNK)], inds_v)

    def step(k, carry):
        idx = inds_v[pl.ds(k * _LANES, _LANES)]
        flg = plsc.load_gather(flags_v, [idx >> 7, idx & 127])
        m = 1 - flg
        mask_v[pl.ds(k * _LANES, _LANES)] = m
        lcsum_v[pl.ds(k * _LANES, _LANES)] = plsc.cumsum(m) + carry
        return carry + jnp.sum(m)

    total = lax.fori_loop(0, CHUNK // _LANES, step, jnp.int32(0))

    tot128_v[pl.ds(0, _LANES)] = jnp.where(lanes == 0, total, 0)
    for t in range(1, 8):
        tot128_v[pl.ds(t * _LANES, _LANES)] = jnp.zeros((_LANES,), jnp.int32)
    pltpu.sync_copy(mask_v, mask_hbm.at[pl.ds(base, CHUNK)])
    pltpu.sync_copy(lcsum_v, lcsum_hbm.at[pl.ds(base, CHUNK)])
    pltpu.sync_copy(tot128_v, totals_hbm.at[wid])


def _mask_scan(flags, inds):
    mesh = plsc.VectorSubcoreMesh(core_axis_name="c", subcore_axis_name="s")
    return pl.kernel(
        _maskscan_body,
        compiler_params=pltpu.CompilerParams(needs_layout_passes=False, use_tc_tiling_on_sc=False),
        out_type=(
            jax.ShapeDtypeStruct((N,), jnp.int32),
            jax.ShapeDtypeStruct((N,), jnp.int32),
            jax.ShapeDtypeStruct((NW, 128), jnp.int32),
        ),
        mesh=mesh,
        scratch_types=[
            pltpu.VMEM((M // 128, 128), jnp.int32),
            pltpu.VMEM((CHUNK,), jnp.int32),
            pltpu.VMEM((CHUNK,), jnp.int32),
            pltpu.VMEM((CHUNK,), jnp.int32),
            pltpu.VMEM((128,), jnp.int32),
        ],
    )(flags, inds)


def _prelude(mask_hbm, lcsum_hbm, totals_hbm, mask_v, lcsum_v, tot_v,
             dest_store):
    cid = lax.axis_index("c")
    sid = lax.axis_index("s")
    wid = cid * 16 + sid
    base = wid * CHUNK
    lanes = jnp.arange(_LANES, dtype=jnp.int32)
    zeros16 = jnp.zeros((_LANES,), jnp.int32)

    pltpu.sync_copy(mask_hbm.at[pl.ds(base, CHUNK)], mask_v)
    pltpu.sync_copy(lcsum_hbm.at[pl.ds(base, CHUNK)], lcsum_v)
    pltpu.sync_copy(totals_hbm, tot_v)

    t_lo = plsc.load_gather(tot_v, [lanes, zeros16])
    t_hi = plsc.load_gather(tot_v, [lanes + 16, zeros16])
    zero_v = jnp.zeros((_LANES,), jnp.int32)
    my_pre = (jnp.sum(jnp.where(lanes < wid, t_lo, zero_v))
              + jnp.sum(jnp.where(lanes + 16 < wid, t_hi, zero_v)))
    tot_all = jnp.sum(t_lo) + jnp.sum(t_hi)

    @pl.loop(0, CHUNK // _LANES)
    def _dest(k):
        m = mask_v[pl.ds(k * _LANES, _LANES)]
        cs = lcsum_v[pl.ds(k * _LANES, _LANES)]
        gi = cs + my_pre
        ivec = base + k * _LANES + lanes
        d = jnp.where(m == 1, gi - 1, tot_all + ivec - gi)
        dest_store(k, d)

    return wid, base


def _xyz_body(mask_hbm, lcsum_hbm, totals_hbm,
              px_hbm, py_hbm, pz_hbm, c1_hbm, c2_hbm, c3_hbm,
              xs_out, ys_out, zs_out,
              mask_v, lcsum_v, tot_v, dest1_v, sl_v, x_v, y_v, z_v, sem):
    def dest_store(k, d):
        dest1_v[pl.ds(k * _LANES, _LANES)] = d

    wid, base = _prelude(mask_hbm, lcsum_hbm, totals_hbm,
                         mask_v, lcsum_v, tot_v, dest_store)
    slice_srcs = (px_hbm, py_hbm, pz_hbm, c1_hbm, c2_hbm, c3_hbm)
    for r in range(6):
        pltpu.sync_copy(slice_srcs[r].at[pl.ds(base, CHUNK)], sl_v.at[r])

    @pl.loop(0, CHUNK // _LANES)
    def _cmp(k):
        s = k * _LANES
        px = sl_v[0, pl.ds(s, _LANES)]
        py = sl_v[1, pl.ds(s, _LANES)]
        pz = sl_v[2, pl.ds(s, _LANES)]
        c1 = sl_v[3, pl.ds(s, _LANES)]
        c2 = sl_v[4, pl.ds(s, _LANES)]
        c3 = sl_v[5, pl.ds(s, _LANES)]
        x_v[pl.ds(s, _LANES)] = (
            px - ((c3 + 0.5) * VOXEL_SIZE[0] + PC_MIN[0]))
        y_v[pl.ds(s, _LANES)] = (
            py - ((c2 + 0.5) * VOXEL_SIZE[1] + PC_MIN[1]))
        z_v[pl.ds(s, _LANES)] = (
            pz - ((c1 + 0.5) * VOXEL_SIZE[2] + PC_MIN[2]))

    pltpu.async_copy(x_v, xs_out.at[dest1_v], sem)
    pltpu.async_copy(y_v, ys_out.at[dest1_v], sem)
    pltpu.async_copy(z_v, zs_out.at[dest1_v], sem)
    pltpu.make_async_copy(x_v, xs_out.at[dest1_v], sem).wait()
    pltpu.make_async_copy(y_v, ys_out.at[dest1_v], sem).wait()
    pltpu.make_async_copy(z_v, zs_out.at[dest1_v], sem).wait()


def _xyz_pass(mask, lcsum, totals, cols):
    mesh = plsc.VectorSubcoreMesh(core_axis_name="c", subcore_axis_name="s")
    return pl.kernel(
        _xyz_body,
        compiler_params=pltpu.CompilerParams(needs_layout_passes=False, use_tc_tiling_on_sc=False),
        out_type=(
            jax.ShapeDtypeStruct((N,), jnp.float32),
            jax.ShapeDtypeStruct((N,), jnp.float32),
            jax.ShapeDtypeStruct((N,), jnp.float32),
        ),
        mesh=mesh,
        scratch_types=(
            [pltpu.VMEM((CHUNK,), jnp.int32)] * 2
            + [pltpu.VMEM((NW, 128), jnp.int32)]
            + [pltpu.VMEM((CHUNK,), jnp.int32)]
            + [pltpu.VMEM((6, CHUNK), jnp.float32)]
            + [pltpu.VMEM((CHUNK,), jnp.float32)] * 3
            + [pltpu.SemaphoreType.DMA]
        ),
    )(mask, lcsum, totals, *cols)


def _feats_body(vfeats_hbm, inds2_hbm, mask_hbm, lcsum_hbm, totals_hbm,
                feats_out,
                mask_v, lcsum_v, tot_v, dest2_v, inds2_v,
                f0, f1, f2, f3, g0, g1, g2, g3, s0, s1, s2, s3):
    feats_bufs = (f0, f1, f2, f3)
    gsems = (g0, g1, g2, g3)
    ssems = (s0, s1, s2, s3)

    def dest_store(k, d):
        dest2_v[k >> 3, pl.ds((k & 7) * _LANES, _LANES)] = d

    wid, base = _prelude(mask_hbm, lcsum_hbm, totals_hbm,
                         mask_v, lcsum_v, tot_v, dest_store)
    pltpu.sync_copy(inds2_hbm.at[pl.ds(wid * NSUB, NSUB)], inds2_v)

    def fire_gather(j, b):
        pltpu.async_copy(vfeats_hbm.at[inds2_v.at[j]], feats_bufs[b], gsems[b])

    def drain_gather(b):
        pltpu.make_async_copy(vfeats_hbm.at[inds2_v.at[0]], feats_bufs[b],
                              gsems[b]).wait()

    def fire_scatter(j, b):
        pltpu.async_copy(feats_bufs[b], feats_out.at[dest2_v.at[j]], ssems[b])

    def drain_scatter(b):
        pltpu.make_async_copy(feats_bufs[b], feats_out.at[dest2_v.at[0]],
                              ssems[b]).wait()

    fire_gather(0, 0)
    fire_gather(1, 1)

    @pl.loop(0, NSUB, step=4)
    def _ring(g):
        for b in range(4):
            j = g + b
            bp = (b + 2) % 4

            @pl.when(jnp.logical_and(j >= 2, j + 2 < NSUB))
            def _():
                drain_scatter(bp)

            @pl.when(j + 2 < NSUB)
            def _():
                fire_gather(j + 2, bp)

            drain_gather(b)
            fire_scatter(j, b)

    drain_scatter(0)
    drain_scatter(1)
    drain_scatter(2)
    drain_scatter(3)


def _feats_pass(voxel_feats, inds2, mask, lcsum, totals):
    mesh = plsc.VectorSubcoreMesh(core_axis_name="c", subcore_axis_name="s")
    return pl.kernel(
        _feats_body,
        compiler_params=pltpu.CompilerParams(needs_layout_passes=False, use_tc_tiling_on_sc=True),
        out_type=jax.ShapeDtypeStruct((N, C), jnp.float32),
        mesh=mesh,
        scratch_types=(
            [pltpu.VMEM((CHUNK,), jnp.int32)] * 2
            + [pltpu.VMEM((NW, 128), jnp.int32)]
            + [pltpu.VMEM((NSUB, SUB), jnp.int32)] * 2
            + [pltpu.VMEM((SUB, C), jnp.float32)] * 4
            + [pltpu.SemaphoreType.DMA] * 8
        ),
    )(voxel_feats, inds2, mask, lcsum, totals)


def _assemble_body(f_ref, x_ref, y_ref, z_ref, o_ref):
    o_ref[...] = jnp.concatenate(
        [f_ref[...], x_ref[...][:, None], y_ref[...][:, None],
         z_ref[...][:, None]], axis=1)


def _assemble(feats_s, xs, ys, zs):
    BR = 1024
    return pl.pallas_call(
        _assemble_body,
        grid=(N // BR,),
        in_specs=[
            pl.BlockSpec((BR, C), lambda i: (i, 0)),
            pl.BlockSpec((BR,), lambda i: (i,)),
            pl.BlockSpec((BR,), lambda i: (i,)),
            pl.BlockSpec((BR,), lambda i: (i,)),
        ],
        out_specs=pl.BlockSpec((BR, C + 3), lambda i: (i, 0)),
        out_shape=jax.ShapeDtypeStruct((N, C + 3), jnp.float32),
    )(feats_s, xs, ys, zs)


def kernel(points, pts_coors, voxel_feats, voxel2point_inds, voxel_padding):
    pad = jnp.asarray(voxel_padding, jnp.float32).reshape(1, 1)
    flags = _compute_flags(pad, voxel_feats)
    mask_i32, lcsum, totals = _mask_scan(flags.reshape(M // 128, 128),
                                         voxel2point_inds)
    inds2 = voxel2point_inds.reshape(N // SUB, SUB)
    cols = (points[:, 0], points[:, 1], points[:, 2],
            pts_coors[:, 1].astype(jnp.float32),
            pts_coors[:, 2].astype(jnp.float32),
            pts_coors[:, 3].astype(jnp.float32))
    xs, ys, zs = _xyz_pass(mask_i32, lcsum, totals, cols)
    feats_s = _feats_pass(voxel_feats, inds2, mask_i32, lcsum, totals)
    results = _assemble(feats_s, xs, ys, zs)
    return results, mask_i32.astype(bool)


# speedup vs baseline: 4.2960x; 3.8076x over previous
import jax
import jax.numpy as jnp
from jax import lax
from jax.experimental import pallas as pl
from jax.experimental.pallas import tpu as pltpu
from jax.experimental.pallas import tpu_sc as plsc

N = 262144
M = 65536
C = 128
NW = 32
CHUNK = N // NW
SUB = 128
NSUB = CHUNK // SUB

VOXEL_SIZE = (1.0, 1.0, 0.08)
PC_MIN = (-50.0, -50.0, -5.0)

_LANES = 16


def _flags_body(pad_ref, feats_ref, flags_ref):
    pad = pad_ref[0, 0]
    flags_ref[...] = jnp.all(feats_ref[...] == pad, axis=1).astype(jnp.int32)


def _compute_flags(pad, voxel_feats):
    BM = 1024
    return pl.pallas_call(
        _flags_body,
        grid=(M // BM,),
        in_specs=[
            pl.BlockSpec(memory_space=pltpu.SMEM),
            pl.BlockSpec((BM, C), lambda i: (i, 0)),
        ],
        out_specs=pl.BlockSpec((BM,), lambda i: (i,)),
        out_shape=jax.ShapeDtypeStruct((M,), jnp.int32),
    )(pad, voxel_feats)


def _maskscan_body(flags_hbm, inds_hbm, mask_hbm, lcsum_hbm, totals_hbm,
                   flags_v, inds_v, mask_v, lcsum_v, tot128_v):
    cid = lax.axis_index("c")
    sid = lax.axis_index("s")
    wid = cid * 16 + sid
    base = wid * CHUNK
    lanes = jnp.arange(_LANES, dtype=jnp.int32)

    pltpu.sync_copy(flags_hbm, flags_v)
    pltpu.sync_copy(inds_hbm.at[pl.ds(base, CHUNK)], inds_v)

    def step(k, carry):
        idx = inds_v[pl.ds(k * _LANES, _LANES)]
        flg = plsc.load_gather(flags_v, [idx >> 7, idx & 127])
        m = 1 - flg
        mask_v[pl.ds(k * _LANES, _LANES)] = m
        lcsum_v[pl.ds(k * _LANES, _LANES)] = plsc.cumsum(m) + carry
        return carry + jnp.sum(m)

    total = lax.fori_loop(0, CHUNK // _LANES, step, jnp.int32(0))

    tot128_v[pl.ds(0, _LANES)] = jnp.where(lanes == 0, total, 0)
    for t in range(1, 8):
        tot128_v[pl.ds(t * _LANES, _LANES)] = jnp.zeros((_LANES,), jnp.int32)
    pltpu.sync_copy(mask_v, mask_hbm.at[pl.ds(base, CHUNK)])
    pltpu.sync_copy(lcsum_v, lcsum_hbm.at[pl.ds(base, CHUNK)])
    pltpu.sync_copy(tot128_v, totals_hbm.at[wid])


def _mask_scan(flags, inds):
    mesh = plsc.VectorSubcoreMesh(core_axis_name="c", subcore_axis_name="s")
    return pl.kernel(
        _maskscan_body,
        compiler_params=pltpu.CompilerParams(needs_layout_passes=False, use_tc_tiling_on_sc=False),
        out_type=(
            jax.ShapeDtypeStruct((N,), jnp.int32),
            jax.ShapeDtypeStruct((N,), jnp.int32),
            jax.ShapeDtypeStruct((NW, 128), jnp.int32),
        ),
        mesh=mesh,
        scratch_types=[
            pltpu.VMEM((M // 128, 128), jnp.int32),
            pltpu.VMEM((CHUNK,), jnp.int32),
            pltpu.VMEM((CHUNK,), jnp.int32),
            pltpu.VMEM((CHUNK,), jnp.int32),
            pltpu.VMEM((128,), jnp.int32),
        ],
    )(flags, inds)


def _prelude(mask_hbm, lcsum_hbm, totals_hbm, mask_v, lcsum_v, tot_v,
             dest_store):
    cid = lax.axis_index("c")
    sid = lax.axis_index("s")
    wid = cid * 16 + sid
    base = wid * CHUNK
    lanes = jnp.arange(_LANES, dtype=jnp.int32)
    zeros16 = jnp.zeros((_LANES,), jnp.int32)

    pltpu.sync_copy(mask_hbm.at[pl.ds(base, CHUNK)], mask_v)
    pltpu.sync_copy(lcsum_hbm.at[pl.ds(base, CHUNK)], lcsum_v)
    pltpu.sync_copy(totals_hbm, tot_v)

    t_lo = plsc.load_gather(tot_v, [lanes, zeros16])
    t_hi = plsc.load_gather(tot_v, [lanes + 16, zeros16])
    zero_v = jnp.zeros((_LANES,), jnp.int32)
    my_pre = (jnp.sum(jnp.where(lanes < wid, t_lo, zero_v))
              + jnp.sum(jnp.where(lanes + 16 < wid, t_hi, zero_v)))
    tot_all = jnp.sum(t_lo) + jnp.sum(t_hi)

    @pl.loop(0, CHUNK // _LANES)
    def _dest(k):
        m = mask_v[pl.ds(k * _LANES, _LANES)]
        cs = lcsum_v[pl.ds(k * _LANES, _LANES)]
        gi = cs + my_pre
        ivec = base + k * _LANES + lanes
        d = jnp.where(m == 1, gi - 1, tot_all + ivec - gi)
        dest_store(k, d)

    return wid, base


def _xyz_body(mask_hbm, lcsum_hbm, totals_hbm,
              px_hbm, py_hbm, pz_hbm, c1_hbm, c2_hbm, c3_hbm,
              xyz8_out,
              mask_v, lcsum_v, tot_v, dest1_v, sl_v, x8_v, sem):
    def dest_store(k, d):
        dest1_v[pl.ds(k * _LANES, _LANES)] = d

    wid, base = _prelude(mask_hbm, lcsum_hbm, totals_hbm,
                         mask_v, lcsum_v, tot_v, dest_store)
    slice_srcs = (px_hbm, py_hbm, pz_hbm, c1_hbm, c2_hbm, c3_hbm)
    lanes = jnp.arange(_LANES, dtype=jnp.int32)
    zeros16 = jnp.zeros((_LANES,), jnp.int32)
    H = CHUNK // 2

    for h in range(2):
        for r in range(6):
            pltpu.sync_copy(slice_srcs[r].at[pl.ds(base + h * H, H)],
                            sl_v.at[r])

        @pl.loop(0, H // _LANES)
        def _cmp(k):
            s = k * _LANES
            px = sl_v[0, pl.ds(s, _LANES)]
            py = sl_v[1, pl.ds(s, _LANES)]
            pz = sl_v[2, pl.ds(s, _LANES)]
            c1 = sl_v[3, pl.ds(s, _LANES)]
            c2 = sl_v[4, pl.ds(s, _LANES)]
            c3 = sl_v[5, pl.ds(s, _LANES)]
            rows = h * H + s + lanes
            plsc.store_scatter(
                x8_v, [rows, zeros16],
                px - ((c3 + 0.5) * VOXEL_SIZE[0] + PC_MIN[0]))
            plsc.store_scatter(
                x8_v, [rows, zeros16 + 1],
                py - ((c2 + 0.5) * VOXEL_SIZE[1] + PC_MIN[1]))
            plsc.store_scatter(
                x8_v, [rows, zeros16 + 2],
                pz - ((c1 + 0.5) * VOXEL_SIZE[2] + PC_MIN[2]))

    pltpu.async_copy(x8_v, xyz8_out.at[dest1_v], sem)
    pltpu.make_async_copy(x8_v, xyz8_out.at[dest1_v], sem).wait()


def _xyz_pass(mask, lcsum, totals, cols):
    mesh = plsc.VectorSubcoreMesh(core_axis_name="c", subcore_axis_name="s")
    return pl.kernel(
        _xyz_body,
        compiler_params=pltpu.CompilerParams(needs_layout_passes=False, use_tc_tiling_on_sc=False),
        out_type=jax.ShapeDtypeStruct((N, 8), jnp.float32),
        mesh=mesh,
        scratch_types=(
            [pltpu.VMEM((CHUNK,), jnp.int32)] * 2
            + [pltpu.VMEM((NW, 128), jnp.int32)]
            + [pltpu.VMEM((CHUNK,), jnp.int32)]
            + [pltpu.VMEM((6, CHUNK // 2), jnp.float32)]
            + [pltpu.VMEM((CHUNK, 8), jnp.float32)]
            + [pltpu.SemaphoreType.DMA]
        ),
    )(mask, lcsum, totals, *cols)


def _feats_body(vfeats_hbm, inds2_hbm, mask_hbm, lcsum_hbm, totals_hbm,
                feats_out,
                mask_v, lcsum_v, tot_v, dest2_v, inds2_v,
                f0, f1, f2, f3, g0, g1, g2, g3, s0, s1, s2, s3):
    feats_bufs = (f0, f1, f2, f3)
    gsems = (g0, g1, g2, g3)
    ssems = (s0, s1, s2, s3)

    def dest_store(k, d):
        dest2_v[k >> 3, pl.ds((k & 7) * _LANES, _LANES)] = d

    wid, base = _prelude(mask_hbm, lcsum_hbm, totals_hbm,
                         mask_v, lcsum_v, tot_v, dest_store)
    pltpu.sync_copy(inds2_hbm.at[pl.ds(wid * NSUB, NSUB)], inds2_v)

    def fire_gather(j, b):
        pltpu.async_copy(vfeats_hbm.at[inds2_v.at[j]], feats_bufs[b], gsems[b])

    def drain_gather(b):
        pltpu.make_async_copy(vfeats_hbm.at[inds2_v.at[0]], feats_bufs[b],
                              gsems[b]).wait()

    def fire_scatter(j, b):
        pltpu.async_copy(feats_bufs[b], feats_out.at[dest2_v.at[j]], ssems[b])

    def drain_scatter(b):
        pltpu.make_async_copy(feats_bufs[b], feats_out.at[dest2_v.at[0]],
                              ssems[b]).wait()

    fire_gather(0, 0)
    fire_gather(1, 1)

    @pl.loop(0, NSUB, step=4)
    def _ring(g):
        for b in range(4):
            j = g + b
            bp = (b + 2) % 4

            @pl.when(jnp.logical_and(j >= 2, j + 2 < NSUB))
            def _():
                drain_scatter(bp)

            @pl.when(j + 2 < NSUB)
            def _():
                fire_gather(j + 2, bp)

            drain_gather(b)
            fire_scatter(j, b)

    drain_scatter(0)
    drain_scatter(1)
    drain_scatter(2)
    drain_scatter(3)


def _feats_pass(voxel_feats, inds2, mask, lcsum, totals):
    mesh = plsc.VectorSubcoreMesh(core_axis_name="c", subcore_axis_name="s")
    return pl.kernel(
        _feats_body,
        compiler_params=pltpu.CompilerParams(needs_layout_passes=False, use_tc_tiling_on_sc=True),
        out_type=jax.ShapeDtypeStruct((N, C), jnp.float32),
        mesh=mesh,
        scratch_types=(
            [pltpu.VMEM((CHUNK,), jnp.int32)] * 2
            + [pltpu.VMEM((NW, 128), jnp.int32)]
            + [pltpu.VMEM((NSUB, SUB), jnp.int32)] * 2
            + [pltpu.VMEM((SUB, C), jnp.float32)] * 4
            + [pltpu.SemaphoreType.DMA] * 8
        ),
    )(voxel_feats, inds2, mask, lcsum, totals)


_BR = 2048


def _assemble_body(f_ref, x_ref, o_ref):
    o_ref[...] = jnp.concatenate([f_ref[...], x_ref[:, :3]], axis=1)


def _assemble(feats_s, xyz8):
    return pl.pallas_call(
        _assemble_body,
        grid=(N // _BR,),
        in_specs=[
            pl.BlockSpec((_BR, C), lambda i: (i, 0)),
            pl.BlockSpec((_BR, 8), lambda i: (i, 0)),
        ],
        out_specs=pl.BlockSpec((_BR, C + 3), lambda i: (i, 0)),
        out_shape=jax.ShapeDtypeStruct((N, C + 3), jnp.float32),
    )(feats_s, xyz8)


def kernel(points, pts_coors, voxel_feats, voxel2point_inds, voxel_padding):
    pad = jnp.asarray(voxel_padding, jnp.float32).reshape(1, 1)
    flags = _compute_flags(pad, voxel_feats)
    mask_i32, lcsum, totals = _mask_scan(flags.reshape(M // 128, 128),
                                         voxel2point_inds)
    inds2 = voxel2point_inds.reshape(N // SUB, SUB)
    cols = (points[:, 0], points[:, 1], points[:, 2],
            pts_coors[:, 1].astype(jnp.float32),
            pts_coors[:, 2].astype(jnp.float32),
            pts_coors[:, 3].astype(jnp.float32))
    xyz8 = _xyz_pass(mask_i32, lcsum, totals, cols)
    feats_s = _feats_pass(voxel_feats, inds2, mask_i32, lcsum, totals)
    results = _assemble(feats_s, xyz8)
    return results, mask_i32.astype(bool)
